# Initial kernel scaffold; baseline (speedup 1.0000x reference)
import functools

import jax
import jax.numpy as jnp
from jax import lax
from jax.experimental import pallas as pl
from jax.experimental.pallas import tpu as pltpu
from jax.experimental.pallas import tpu_sc as plsc

B = 16384
D = 64
HID = 64
NC, NS = 2, 16
NW = NC * NS
BPW = B // NW
CH = 128
NCH = BPW // CH

_mesh = plsc.VectorSubcoreMesh(core_axis_name="c", subcore_axis_name="s")


@functools.partial(
    pl.kernel,
    out_type=(
        jax.ShapeDtypeStruct((B, D), jnp.float32),
        jax.ShapeDtypeStruct((B, D), jnp.float32),
    ),
    mesh=_mesh,
    scratch_types=[
        pltpu.VMEM((BPW,), jnp.int32),
        pltpu.VMEM((BPW,), jnp.int32),
        pltpu.VMEM((BPW,), jnp.int32),
        pltpu.VMEM((BPW,), jnp.int32),
        pltpu.VMEM((BPW, D), jnp.float32),
        pltpu.VMEM((BPW, D), jnp.float32),
        pltpu.SemaphoreType.DMA,
        pltpu.SemaphoreType.DMA,
    ],
)
def _sc_gather(sim_hbm, midx_hbm, didx_hbm, em_hbm, ed_hbm,
               mfea_hbm, dfea_hbm,
               eidx_m, eidx_d, cidx_m, cidx_d, rows_m, rows_d,
               sem_m, sem_d):
    wid = lax.axis_index("s") * NC + lax.axis_index("c")
    base = wid * BPW
    pltpu.sync_copy(midx_hbm.at[pl.ds(base, BPW)], eidx_m)
    pltpu.sync_copy(didx_hbm.at[pl.ds(base, BPW)], eidx_d)
    pend = []
    for j in range(NCH):
        s = pl.ds(j * CH, CH)
        pend.append(pltpu.async_copy(sim_hbm.at[eidx_m.at[s]], cidx_m.at[s], sem_m))
        pend.append(pltpu.async_copy(sim_hbm.at[eidx_d.at[s]], cidx_d.at[s], sem_d))
    for h in pend:
        h.wait()
    pend = []
    for j in range(NCH):
        s = pl.ds(j * CH, CH)
        pend.append(pltpu.async_copy(em_hbm.at[cidx_m.at[s]], rows_m.at[s], sem_m))
        pend.append(pltpu.async_copy(ed_hbm.at[cidx_d.at[s]], rows_d.at[s], sem_d))
    for h in pend:
        h.wait()
    pltpu.sync_copy(rows_m, mfea_hbm.at[pl.ds(base, BPW)])
    pltpu.sync_copy(rows_d, dfea_hbm.at[pl.ds(base, BPW)])


BLK = 2048


def _mlp_body(mfea_ref, dfea_ref, w1m_ref, w1d_ref, b1_ref, w2_ref, b2_ref,
              out_ref):
    h = (
        jnp.dot(mfea_ref[...], w1m_ref[...],
                preferred_element_type=jnp.float32,
                precision=lax.Precision.HIGHEST)
        + jnp.dot(dfea_ref[...], w1d_ref[...],
                  preferred_element_type=jnp.float32,
                  precision=lax.Precision.HIGHEST)
        + b1_ref[...]
    )
    h = jnp.maximum(h, 0.0)
    z = jnp.sum(h * w2_ref[...], axis=1) + b2_ref[0, 0]
    out_ref[...] = jax.nn.sigmoid(z)


_mlp = pl.pallas_call(
    _mlp_body,
    grid=(B // BLK,),
    in_specs=[
        pl.BlockSpec((BLK, D), lambda i: (i, 0)),
        pl.BlockSpec((BLK, D), lambda i: (i, 0)),
        pl.BlockSpec((D, HID), lambda i: (0, 0)),
        pl.BlockSpec((D, HID), lambda i: (0, 0)),
        pl.BlockSpec((1, HID), lambda i: (0, 0)),
        pl.BlockSpec((1, HID), lambda i: (0, 0)),
        pl.BlockSpec((1, 1), lambda i: (0, 0)),
    ],
    out_specs=pl.BlockSpec((BLK,), lambda i: (i,)),
    out_shape=jax.ShapeDtypeStruct((B,), jnp.float32),
)


def kernel(sim_data, train_data, Em_table, Ed_table, W1, b1, W2, b2):
    m_index = train_data[:, 0]
    d_index = train_data[:, 1]
    mfea, dfea = _sc_gather(sim_data, m_index, d_index, Em_table, Ed_table)
    w1m = W1[:D]
    w1d = W1[D:]
    return _mlp(mfea, dfea, w1m, w1d, b1.reshape(1, HID), W2.reshape(1, HID),
                b2.reshape(1, 1))


# speedup vs baseline: 3.3173x; 3.3173x over previous
import functools

import jax
import jax.numpy as jnp
from jax import lax
from jax.experimental import pallas as pl
from jax.experimental.pallas import tpu as pltpu
from jax.experimental.pallas import tpu_sc as plsc

B = 16384
D = 64
HID = 64
NC, NS = 2, 16
NW = NC * NS
BPW = B // NW
CH = 128
NCH = BPW // CH

_mesh = plsc.VectorSubcoreMesh(core_axis_name="c", subcore_axis_name="s")


@functools.partial(
    pl.kernel,
    out_type=(
        jax.ShapeDtypeStruct((B, D), jnp.float32),
        jax.ShapeDtypeStruct((B, D), jnp.float32),
    ),
    mesh=_mesh,
    scratch_types=[
        pltpu.VMEM((BPW,), jnp.int32),
        pltpu.VMEM((BPW,), jnp.int32),
        pltpu.VMEM((BPW,), jnp.int32),
        pltpu.VMEM((BPW,), jnp.int32),
        pltpu.VMEM((BPW, D), jnp.float32),
        pltpu.VMEM((BPW, D), jnp.float32),
        pltpu.SemaphoreType.DMA,
        pltpu.SemaphoreType.DMA,
    ],
    compiler_params=pltpu.CompilerParams(use_tc_tiling_on_sc=False),
)
def _sc_gather(sim_hbm, midx_hbm, didx_hbm, em_hbm, ed_hbm,
               mfea_hbm, dfea_hbm,
               eidx_m, eidx_d, cidx_m, cidx_d, rows_m, rows_d,
               sem_m, sem_d):
    wid = lax.axis_index("s") * NC + lax.axis_index("c")
    base = wid * BPW
    pltpu.sync_copy(midx_hbm.at[pl.ds(base, BPW)], eidx_m)
    pltpu.sync_copy(didx_hbm.at[pl.ds(base, BPW)], eidx_d)
    pend = []
    for j in range(NCH):
        s = pl.ds(j * CH, CH)
        pend.append(pltpu.async_copy(sim_hbm.at[eidx_m.at[s]], cidx_m.at[s], sem_m))
        pend.append(pltpu.async_copy(sim_hbm.at[eidx_d.at[s]], cidx_d.at[s], sem_d))
    for h in pend:
        h.wait()
    pend = []
    for j in range(NCH):
        s = pl.ds(j * CH, CH)
        pend.append(pltpu.async_copy(em_hbm.at[cidx_m.at[s]], rows_m.at[s], sem_m))
        pend.append(pltpu.async_copy(ed_hbm.at[cidx_d.at[s]], rows_d.at[s], sem_d))
    for h in pend:
        h.wait()
    pltpu.sync_copy(rows_m, mfea_hbm.at[pl.ds(base, BPW)])
    pltpu.sync_copy(rows_d, dfea_hbm.at[pl.ds(base, BPW)])


BLK = 2048


def _mlp_body(mfea_ref, dfea_ref, w1m_ref, w1d_ref, b1_ref, w2_ref, b2_ref,
              out_ref):
    h = (
        jnp.dot(mfea_ref[...], w1m_ref[...],
                preferred_element_type=jnp.float32,
                precision=lax.Precision.HIGHEST)
        + jnp.dot(dfea_ref[...], w1d_ref[...],
                  preferred_element_type=jnp.float32,
                  precision=lax.Precision.HIGHEST)
        + b1_ref[...]
    )
    h = jnp.maximum(h, 0.0)
    z = jnp.sum(h * w2_ref[...], axis=1) + b2_ref[0, 0]
    out_ref[...] = jax.nn.sigmoid(z)


_mlp = pl.pallas_call(
    _mlp_body,
    grid=(B // BLK,),
    in_specs=[
        pl.BlockSpec((BLK, D), lambda i: (i, 0)),
        pl.BlockSpec((BLK, D), lambda i: (i, 0)),
        pl.BlockSpec((D, HID), lambda i: (0, 0)),
        pl.BlockSpec((D, HID), lambda i: (0, 0)),
        pl.BlockSpec((1, HID), lambda i: (0, 0)),
        pl.BlockSpec((1, HID), lambda i: (0, 0)),
        pl.BlockSpec((1, 1), lambda i: (0, 0)),
    ],
    out_specs=pl.BlockSpec((BLK,), lambda i: (i,)),
    out_shape=jax.ShapeDtypeStruct((B,), jnp.float32),
)


def kernel(sim_data, train_data, Em_table, Ed_table, W1, b1, W2, b2):
    m_index = train_data[:, 0]
    d_index = train_data[:, 1]
    mfea, dfea = _sc_gather(sim_data, m_index, d_index, Em_table, Ed_table)
    w1m = W1[:D]
    w1d = W1[D:]
    return _mlp(mfea, dfea, w1m, w1d, b1.reshape(1, HID), W2.reshape(1, HID),
                b2.reshape(1, 1))
